# baseline (speedup 1.0000x reference)
import jax
import jax.numpy as jnp
from jax import lax
from jax.experimental import pallas as pl
from jax.experimental.pallas import tpu as pltpu
from jax.experimental.pallas import tpu_sc as plsc

N = 10000
NP = 10240
E = 320000
K = 128
ROWSP = 2560
EP = ROWSP * K
NSUB = 16
NW = 32
SB = 16
RPS = ROWSP // NSUB
RPSE = ROWSP // NW
NPS = NP // NSUB
EPD = 327680
ECH = 2560
EPW = EPD // NW

_MESH = plsc.VectorSubcoreMesh(core_axis_name="c", subcore_axis_name="s")


def _deg_body(dstf, degp, hist, dbuf):
    cid = lax.axis_index("c")
    sid = lax.axis_index("s")
    wid = cid * NSUB + sid

    def zero(i, c):
        hist[pl.ds(i * 16, 16)] = jnp.zeros((16,), jnp.float32)
        return c

    lax.fori_loop(0, NP // 16, zero, 0)
    base = wid * EPW
    ones = jnp.ones((16,), jnp.float32)

    def outer(g, carry):
        pltpu.sync_copy(dstf.at[pl.ds(base + g * ECH, ECH)], dbuf)

        def inner(p, c2):
            idx = dbuf[pl.ds(p * 16, 16)]
            plsc.addupdate_scatter(hist, [idx], ones)
            return c2

        return lax.fori_loop(0, ECH // 16, inner, carry)

    lax.fori_loop(0, EPW // ECH, outer, 0)
    pltpu.sync_copy(hist, degp.at[pl.ds(wid * NP, NP)])


_deg_kernel = pl.kernel(
    _deg_body,
    out_type=jax.ShapeDtypeStruct((NW * NP,), jnp.float32),
    mesh=_MESH,
    compiler_params=pltpu.CompilerParams(needs_layout_passes=False),
    scratch_types=[
        pltpu.VMEM((NP,), jnp.float32),
        pltpu.VMEM((ECH,), jnp.int32),
    ],
)


ND = 2


def _edge_sweep(src2, dst2, v, acc, sidx, didx, rows, semg, sems,
                row0, nrefill):
    def outer(g, carry):
        pltpu.sync_copy(src2.at[pl.ds(row0 + g * SB, SB)], sidx)
        pltpu.sync_copy(dst2.at[pl.ds(row0 + g * SB, SB)], didx)
        cp = pltpu.async_copy(v.at[sidx.at[0]], rows[0], semg[0])
        for j in range(1, SB):
            cp_next = pltpu.async_copy(v.at[sidx.at[j]], rows[j % 2],
                                       semg[j % 2])
            cp.wait()
            pltpu.sync_copy(rows[(j - 1) % 2], acc.at[didx.at[j - 1]],
                            add=True)
            cp = cp_next
        cp.wait()
        pltpu.sync_copy(rows[(SB - 1) % 2], acc.at[didx.at[SB - 1]], add=True)
        return carry

    lax.fori_loop(0, nrefill, outer, 0)


_PROP_SCRATCH = (
    [pltpu.VMEM_SHARED((NP, 128), jnp.float32),
     pltpu.VMEM((SB, K), jnp.int32),
     pltpu.VMEM((SB, K), jnp.int32)]
    + [pltpu.VMEM((K, 128), jnp.float32) for _ in range(ND)]
    + [pltpu.SemaphoreType.DMA for _ in range(2 * ND)]
)

_PROP_OUT = [jax.ShapeDtypeStruct((NP, 128), jnp.float32),
             jax.ShapeDtypeStruct((NP, 128), jnp.float32)]


def _prop_col_body(src2, dst2, va, vb, ya, yb, acc, sidx, didx, *bufs):
    rows, semg, sems = bufs[:ND], bufs[ND:2 * ND], bufs[2 * ND:]
    cid = lax.axis_index("c")
    sid = lax.axis_index("s")

    def run(v, y):
        pltpu.sync_copy(v.at[pl.ds(sid * NPS, NPS)],
                        acc.at[pl.ds(sid * NPS, NPS)])
        plsc.subcore_barrier()
        _edge_sweep(src2, dst2, v, acc, sidx, didx, rows,
                    semg, sems, sid * RPS, RPS // SB)
        plsc.subcore_barrier()
        pltpu.sync_copy(acc.at[pl.ds(sid * NPS, NPS)],
                        y.at[pl.ds(sid * NPS, NPS)])

    @pl.when(cid == 0)
    def _():
        run(va, ya)

    @pl.when(cid == 1)
    def _():
        run(vb, yb)


_prop_col = pl.kernel(
    _prop_col_body,
    out_type=_PROP_OUT,
    mesh=_MESH,
    scratch_types=_PROP_SCRATCH,
)


def _prop_edge_body(src2, dst2, va, vb, vz, pa, pb, acc, sidx, didx, *bufs):
    rows, semg, sems = bufs[:ND], bufs[ND:2 * ND], bufs[2 * ND:]
    cid = lax.axis_index("c")
    sid = lax.axis_index("s")

    def run(v, p):
        pltpu.sync_copy(vz.at[pl.ds(sid * NPS, NPS)],
                        acc.at[pl.ds(sid * NPS, NPS)])
        plsc.subcore_barrier()
        _edge_sweep(src2, dst2, v, acc, sidx, didx, rows,
                    semg, sems, (cid * NSUB + sid) * RPSE, RPSE // SB)
        plsc.subcore_barrier()
        pltpu.sync_copy(acc.at[pl.ds(sid * NPS, NPS)],
                        p.at[pl.ds(sid * NPS, NPS)])

    @pl.when(cid == 0)
    def _():
        run(va, pa)

    @pl.when(cid == 1)
    def _():
        run(vb, pb)


_prop_edge = pl.kernel(
    _prop_edge_body,
    out_type=_PROP_OUT,
    mesh=_MESH,
    scratch_types=_PROP_SCRATCH,
)


_R = 1024
_G = NP // _R


def _row_spec(c):
    return pl.BlockSpec((_R, c), lambda i: (i, 0))


def _full_spec(r, c):
    return pl.BlockSpec((r, c), lambda i: (0, 0))


def _tc_a_body(dgt, f, c, wf, wce, wcd, dinv_o, v1a_o, v1b_o, v3b_o):
    deg = jnp.sum(dgt[...], axis=1, keepdims=True) + 1.0
    dinv = lax.rsqrt(deg)
    dinv_o[...] = dinv
    v1a_o[...] = dinv * jnp.dot(f[...], wf[...], preferred_element_type=jnp.float32)
    cc = c[...]
    v1b_o[...] = dinv * jnp.dot(cc, wce[...], preferred_element_type=jnp.float32)
    v3b_o[...] = dinv * jnp.dot(cc, wcd[...], preferred_element_type=jnp.float32)


_tc_a = pl.pallas_call(
    _tc_a_body,
    grid=(_G,),
    in_specs=[_row_spec(NW), _row_spec(128), _row_spec(16),
              _full_spec(128, 128), _full_spec(16, 128), _full_spec(16, 128)],
    out_specs=[_row_spec(1), _row_spec(128), _row_spec(128), _row_spec(128)],
    out_shape=[jax.ShapeDtypeStruct((NP, 1), jnp.float32),
               jax.ShapeDtypeStruct((NP, 128), jnp.float32),
               jax.ShapeDtypeStruct((NP, 128), jnp.float32),
               jax.ShapeDtypeStruct((NP, 128), jnp.float32)],
)


def _tc_b_body(ya, yb, dv, bf, bc, wt, wb, v2_o, v2c_o):
    dinv = dv[...]
    ha = jnp.tanh(dinv * ya[...] + bf[...])
    hb = jnp.tanh(dinv * yb[...] + bc[...])
    v2 = dinv * (jnp.dot(ha, wt[...], preferred_element_type=jnp.float32)
                 + jnp.dot(hb, wb[...], preferred_element_type=jnp.float32))
    v2_o[...] = v2
    v2c_o[...] = v2


_tc_b = pl.pallas_call(
    _tc_b_body,
    grid=(_G,),
    in_specs=[_row_spec(128), _row_spec(128), _row_spec(1),
              _full_spec(1, 128), _full_spec(1, 128),
              _full_spec(128, 128), _full_spec(128, 128)],
    out_specs=[_row_spec(128), _row_spec(128)],
    out_shape=[jax.ShapeDtypeStruct((NP, 128), jnp.float32),
               jax.ShapeDtypeStruct((NP, 128), jnp.float32)],
)


def _tc_c_body(pa, pb, v2, dv, nz, bml, wz, mean_o, logvar_o, z_o, v3a_o):
    dinv = dv[...]
    y2 = dinv * (pa[...] + pb[...] + v2[...]) + bml[...]
    mean = y2[:, :64]
    logvar = y2[:, 64:]
    z = nz[...] * jnp.exp(0.5 * logvar) + mean
    mean_o[...] = mean
    logvar_o[...] = logvar
    z_o[...] = z
    v3a_o[...] = dinv * jnp.dot(z, wz[...], preferred_element_type=jnp.float32)


_tc_c = pl.pallas_call(
    _tc_c_body,
    grid=(_G,),
    in_specs=[_row_spec(128), _row_spec(128), _row_spec(128), _row_spec(1),
              _row_spec(64), _full_spec(1, 128), _full_spec(64, 128)],
    out_specs=[_row_spec(64), _row_spec(64), _row_spec(64), _row_spec(128)],
    out_shape=[jax.ShapeDtypeStruct((NP, 64), jnp.float32),
               jax.ShapeDtypeStruct((NP, 64), jnp.float32),
               jax.ShapeDtypeStruct((NP, 64), jnp.float32),
               jax.ShapeDtypeStruct((NP, 128), jnp.float32)],
)


def _tc_d_body(ya, yb, dv, bz, bc, wt, wb, v4_o, v4c_o):
    dinv = dv[...]
    ha = jnp.tanh(dinv * ya[...] + bz[...])
    hb = jnp.tanh(dinv * yb[...] + bc[...])
    v4 = dinv * (jnp.dot(ha, wt[...], preferred_element_type=jnp.float32)
                 + jnp.dot(hb, wb[...], preferred_element_type=jnp.float32))
    v4_o[...] = v4
    v4c_o[...] = v4


_tc_d = pl.pallas_call(
    _tc_d_body,
    grid=(_G,),
    in_specs=[_row_spec(128), _row_spec(128), _row_spec(1),
              _full_spec(1, 128), _full_spec(1, 128),
              _full_spec(128, 128), _full_spec(128, 128)],
    out_specs=[_row_spec(128), _row_spec(128)],
    out_shape=[jax.ShapeDtypeStruct((NP, 128), jnp.float32),
               jax.ShapeDtypeStruct((NP, 128), jnp.float32)],
)


def _tc_e_body(pa, pb, v4, dv, bo, out_o):
    out_o[...] = dv[...] * (pa[...] + pb[...] + v4[...]) + bo[...]


_tc_e = pl.pallas_call(
    _tc_e_body,
    grid=(_G,),
    in_specs=[_row_spec(128), _row_spec(128), _row_spec(128), _row_spec(1),
              _full_spec(1, 128)],
    out_specs=_row_spec(128),
    out_shape=jax.ShapeDtypeStruct((NP, 128), jnp.float32),
)


def _pad_rows(x):
    return jnp.pad(x, ((0, NP - N), (0, 0)))


def kernel(feature, condition, edge_index, W_f2h, b_f2h, W_ce, b_ce, W_mean,
           b_mean, W_logvar, b_logvar, W_z2h, b_z2h, W_cd, b_cd, W_out, b_out):
    src0 = edge_index[0].astype(jnp.int32)
    dst0 = edge_index[1].astype(jnp.int32)
    src = jnp.concatenate(
        [src0, jnp.zeros((EP - E,), jnp.int32)]).reshape(ROWSP, K)
    trash = N + jnp.arange(EP - E, dtype=jnp.int32) % (NP - N)
    dst = jnp.concatenate([dst0, trash]).reshape(ROWSP, K)
    dstd = jnp.concatenate([dst0, trash[:EPD - E]])

    degp = _deg_kernel(dstd)
    degt = degp.reshape(NW, NP).T

    wml = jnp.concatenate([W_mean, W_logvar], axis=1)
    bml = jnp.concatenate([b_mean, b_logvar]).reshape(1, -1)
    vz = jnp.zeros((NP, 128), jnp.float32)

    featp = _pad_rows(feature)
    condp = _pad_rows(condition)

    dinv, v1a, v1b, v3b = _tc_a(degt, featp, condp, W_f2h, W_ce, W_cd)
    y1a, y1b = _prop_col(src, dst, v1a, v1b)
    v2, v2c = _tc_b(y1a, y1b, dinv, b_f2h.reshape(1, -1), b_ce.reshape(1, -1),
                    wml[:128], wml[128:])
    p2a, p2b = _prop_edge(src, dst, v2, v2c, vz)
    noise = _pad_rows(jax.random.normal(jax.random.key(42), (N, 64), jnp.float32))
    mean, logvar, z, v3a = _tc_c(p2a, p2b, v2, dinv, noise, bml, W_z2h)
    y3a, y3b = _prop_col(src, dst, v3a, v3b)
    v4, v4c = _tc_d(y3a, y3b, dinv, b_z2h.reshape(1, -1), b_cd.reshape(1, -1),
                    W_out[:128], W_out[128:])
    p4a, p4b = _prop_edge(src, dst, v4, v4c, vz)
    out = _tc_e(p4a, p4b, v4, dinv, b_out.reshape(1, -1))
    return (z[:N], mean[:N], logvar[:N], out[:N])


# speedup vs baseline: 1.1322x; 1.1322x over previous
import jax
import jax.numpy as jnp
from jax import lax
from jax.experimental import pallas as pl
from jax.experimental.pallas import tpu as pltpu
from jax.experimental.pallas import tpu_sc as plsc

N = 10000
NP = 10240
E = 320000
K = 128
ROWSP = 2560
EP = ROWSP * K
NSUB = 16
NW = 32
SB = 16
RPS = ROWSP // NSUB
RPSE = ROWSP // NW
NPS = NP // NSUB
EPD = 327680
ECH = 2560
EPW = EPD // NW

_MESH = plsc.VectorSubcoreMesh(core_axis_name="c", subcore_axis_name="s")


def _deg_body(dstf, degp, hist, dbuf):
    cid = lax.axis_index("c")
    sid = lax.axis_index("s")
    wid = cid * NSUB + sid

    def zero(i, c):
        hist[pl.ds(i * 16, 16)] = jnp.zeros((16,), jnp.float32)
        return c

    lax.fori_loop(0, NP // 16, zero, 0)
    base = wid * EPW
    ones = jnp.ones((16,), jnp.float32)

    def outer(g, carry):
        pltpu.sync_copy(dstf.at[pl.ds(base + g * ECH, ECH)], dbuf)

        def inner(p, c2):
            idx = dbuf[pl.ds(p * 16, 16)]
            plsc.addupdate_scatter(hist, [idx], ones)
            return c2

        return lax.fori_loop(0, ECH // 16, inner, carry)

    lax.fori_loop(0, EPW // ECH, outer, 0)
    pltpu.sync_copy(hist, degp.at[pl.ds(wid * NP, NP)])


_deg_kernel = pl.kernel(
    _deg_body,
    out_type=jax.ShapeDtypeStruct((NW * NP,), jnp.float32),
    mesh=_MESH,
    compiler_params=pltpu.CompilerParams(needs_layout_passes=False),
    scratch_types=[
        pltpu.VMEM((NP,), jnp.float32),
        pltpu.VMEM((ECH,), jnp.int32),
    ],
)


ND = 2


def _edge_sweep(src2, dst2, v, acc, sidx, didx, rows, semg, sems,
                row0, nrefill):
    def outer(g, carry):
        pltpu.sync_copy(src2.at[pl.ds(row0 + g * SB, SB)], sidx)
        pltpu.sync_copy(dst2.at[pl.ds(row0 + g * SB, SB)], didx)
        cp = pltpu.async_copy(v.at[sidx.at[0]], rows[0], semg[0])
        for j in range(1, SB):
            cp_next = pltpu.async_copy(v.at[sidx.at[j]], rows[j % 2],
                                       semg[j % 2])
            cp.wait()
            pltpu.sync_copy(rows[(j - 1) % 2], acc.at[didx.at[j - 1]],
                            add=True)
            cp = cp_next
        cp.wait()
        pltpu.sync_copy(rows[(SB - 1) % 2], acc.at[didx.at[SB - 1]], add=True)
        return carry

    lax.fori_loop(0, nrefill, outer, 0)


_PROP_SCRATCH = (
    [pltpu.VMEM_SHARED((NP, 128), jnp.float32),
     pltpu.VMEM((SB, K), jnp.int32),
     pltpu.VMEM((SB, K), jnp.int32)]
    + [pltpu.VMEM((K, 128), jnp.float32) for _ in range(ND)]
    + [pltpu.SemaphoreType.DMA for _ in range(2 * ND)]
)

_PROP_OUT = [jax.ShapeDtypeStruct((NP, 128), jnp.float32),
             jax.ShapeDtypeStruct((NP, 128), jnp.float32)]


def _prop_col_body(src2, dst2, va, vb, ya, yb, acc, sidx, didx, *bufs):
    rows, semg, sems = bufs[:ND], bufs[ND:2 * ND], bufs[2 * ND:]
    cid = lax.axis_index("c")
    sid = lax.axis_index("s")

    def run(v, y):
        pltpu.sync_copy(v.at[pl.ds(sid * NPS, NPS)],
                        acc.at[pl.ds(sid * NPS, NPS)])
        plsc.subcore_barrier()
        _edge_sweep(src2, dst2, v, acc, sidx, didx, rows,
                    semg, sems, sid * RPS, RPS // SB)
        plsc.subcore_barrier()
        pltpu.sync_copy(acc.at[pl.ds(sid * NPS, NPS)],
                        y.at[pl.ds(sid * NPS, NPS)])

    @pl.when(cid == 0)
    def _():
        run(va, ya)

    @pl.when(cid == 1)
    def _():
        run(vb, yb)


_prop_col = pl.kernel(
    _prop_col_body,
    out_type=_PROP_OUT,
    mesh=_MESH,
    scratch_types=_PROP_SCRATCH,
)


def _prop_edge_body(src2, dst2, va, vb, pa, pb, acc, sidx, didx, *bufs):
    rows, semg, sems = bufs[:ND], bufs[ND:2 * ND], bufs[2 * ND:]
    cid = lax.axis_index("c")
    sid = lax.axis_index("s")

    def run(v, p):
        def zrow(r, c):
            for cc in range(8):
                rows[0][r, pl.ds(cc * 16, 16)] = jnp.zeros((16,), jnp.float32)
            return c

        lax.fori_loop(0, K, zrow, 0)
        for i in range(NPS // K):
            pltpu.sync_copy(rows[0], acc.at[pl.ds(sid * NPS + i * K, K)])
        plsc.subcore_barrier()
        _edge_sweep(src2, dst2, v, acc, sidx, didx, rows,
                    semg, sems, (cid * NSUB + sid) * RPSE, RPSE // SB)
        plsc.subcore_barrier()
        pltpu.sync_copy(acc.at[pl.ds(sid * NPS, NPS)],
                        p.at[pl.ds(sid * NPS, NPS)])

    @pl.when(cid == 0)
    def _():
        run(va, pa)

    @pl.when(cid == 1)
    def _():
        run(vb, pb)


_prop_edge = pl.kernel(
    _prop_edge_body,
    out_type=_PROP_OUT,
    mesh=_MESH,
    scratch_types=_PROP_SCRATCH,
)


_R = 1024
_G = NP // _R


def _row_spec(c):
    return pl.BlockSpec((_R, c), lambda i: (i, 0))


def _full_spec(r, c):
    return pl.BlockSpec((r, c), lambda i: (0, 0))


def _tc_a_body(dgt, f, c, wf, wce, wcd, dinv_o, v1a_o, v1b_o, v3b_o):
    deg = jnp.sum(dgt[...], axis=1, keepdims=True) + 1.0
    dinv = lax.rsqrt(deg)
    dinv_o[...] = dinv
    v1a_o[...] = dinv * jnp.dot(f[...], wf[...], preferred_element_type=jnp.float32)
    cc = c[...]
    v1b_o[...] = dinv * jnp.dot(cc, wce[...], preferred_element_type=jnp.float32)
    v3b_o[...] = dinv * jnp.dot(cc, wcd[...], preferred_element_type=jnp.float32)


_tc_a = pl.pallas_call(
    _tc_a_body,
    grid=(_G,),
    in_specs=[_row_spec(NW), _row_spec(128), _row_spec(16),
              _full_spec(128, 128), _full_spec(16, 128), _full_spec(16, 128)],
    out_specs=[_row_spec(1), _row_spec(128), _row_spec(128), _row_spec(128)],
    out_shape=[jax.ShapeDtypeStruct((NP, 1), jnp.float32),
               jax.ShapeDtypeStruct((NP, 128), jnp.float32),
               jax.ShapeDtypeStruct((NP, 128), jnp.float32),
               jax.ShapeDtypeStruct((NP, 128), jnp.float32)],
)


def _tc_b_body(ya, yb, dv, bf, bc, wt, wb, v2_o, v2c_o):
    dinv = dv[...]
    ha = jnp.tanh(dinv * ya[...] + bf[...])
    hb = jnp.tanh(dinv * yb[...] + bc[...])
    v2 = dinv * (jnp.dot(ha, wt[...], preferred_element_type=jnp.float32)
                 + jnp.dot(hb, wb[...], preferred_element_type=jnp.float32))
    v2_o[...] = v2
    v2c_o[...] = v2


_tc_b = pl.pallas_call(
    _tc_b_body,
    grid=(_G,),
    in_specs=[_row_spec(128), _row_spec(128), _row_spec(1),
              _full_spec(1, 128), _full_spec(1, 128),
              _full_spec(128, 128), _full_spec(128, 128)],
    out_specs=[_row_spec(128), _row_spec(128)],
    out_shape=[jax.ShapeDtypeStruct((NP, 128), jnp.float32),
               jax.ShapeDtypeStruct((NP, 128), jnp.float32)],
)


def _tc_c_body(pa, pb, v2, dv, nz, bml, wz, mean_o, logvar_o, z_o, v3a_o):
    dinv = dv[...]
    y2 = dinv * (pa[...] + pb[...] + v2[...]) + bml[...]
    mean = y2[:, :64]
    logvar = y2[:, 64:]
    z = nz[...] * jnp.exp(0.5 * logvar) + mean
    mean_o[...] = mean
    logvar_o[...] = logvar
    z_o[...] = z
    v3a_o[...] = dinv * jnp.dot(z, wz[...], preferred_element_type=jnp.float32)


_tc_c = pl.pallas_call(
    _tc_c_body,
    grid=(_G,),
    in_specs=[_row_spec(128), _row_spec(128), _row_spec(128), _row_spec(1),
              _row_spec(64), _full_spec(1, 128), _full_spec(64, 128)],
    out_specs=[_row_spec(64), _row_spec(64), _row_spec(64), _row_spec(128)],
    out_shape=[jax.ShapeDtypeStruct((NP, 64), jnp.float32),
               jax.ShapeDtypeStruct((NP, 64), jnp.float32),
               jax.ShapeDtypeStruct((NP, 64), jnp.float32),
               jax.ShapeDtypeStruct((NP, 128), jnp.float32)],
)


def _tc_d_body(ya, yb, dv, bz, bc, wt, wb, v4_o, v4c_o):
    dinv = dv[...]
    ha = jnp.tanh(dinv * ya[...] + bz[...])
    hb = jnp.tanh(dinv * yb[...] + bc[...])
    v4 = dinv * (jnp.dot(ha, wt[...], preferred_element_type=jnp.float32)
                 + jnp.dot(hb, wb[...], preferred_element_type=jnp.float32))
    v4_o[...] = v4
    v4c_o[...] = v4


_tc_d = pl.pallas_call(
    _tc_d_body,
    grid=(_G,),
    in_specs=[_row_spec(128), _row_spec(128), _row_spec(1),
              _full_spec(1, 128), _full_spec(1, 128),
              _full_spec(128, 128), _full_spec(128, 128)],
    out_specs=[_row_spec(128), _row_spec(128)],
    out_shape=[jax.ShapeDtypeStruct((NP, 128), jnp.float32),
               jax.ShapeDtypeStruct((NP, 128), jnp.float32)],
)


def _tc_e_body(pa, pb, v4, dv, bo, out_o):
    out_o[...] = dv[...] * (pa[...] + pb[...] + v4[...]) + bo[...]


_tc_e = pl.pallas_call(
    _tc_e_body,
    grid=(_G,),
    in_specs=[_row_spec(128), _row_spec(128), _row_spec(128), _row_spec(1),
              _full_spec(1, 128)],
    out_specs=_row_spec(128),
    out_shape=jax.ShapeDtypeStruct((NP, 128), jnp.float32),
)


def _pad_rows(x):
    return jnp.pad(x, ((0, NP - N), (0, 0)))


def kernel(feature, condition, edge_index, W_f2h, b_f2h, W_ce, b_ce, W_mean,
           b_mean, W_logvar, b_logvar, W_z2h, b_z2h, W_cd, b_cd, W_out, b_out):
    src0 = edge_index[0].astype(jnp.int32)
    dst0 = edge_index[1].astype(jnp.int32)
    src = jnp.concatenate(
        [src0, jnp.zeros((EP - E,), jnp.int32)]).reshape(ROWSP, K)
    dst = jnp.concatenate(
        [dst0, jnp.full((EP - E,), N, jnp.int32)]).reshape(ROWSP, K)
    dstd = jnp.concatenate([dst0, jnp.full((EPD - E,), N, jnp.int32)])

    degp = _deg_kernel(dstd)
    degt = degp.reshape(NW, NP).T

    wml = jnp.concatenate([W_mean, W_logvar], axis=1)
    bml = jnp.concatenate([b_mean, b_logvar]).reshape(1, -1)

    featp = _pad_rows(feature)
    condp = _pad_rows(condition)

    dinv, v1a, v1b, v3b = _tc_a(degt, featp, condp, W_f2h, W_ce, W_cd)
    y1a, y1b = _prop_col(src, dst, v1a, v1b)
    v2, v2c = _tc_b(y1a, y1b, dinv, b_f2h.reshape(1, -1), b_ce.reshape(1, -1),
                    wml[:128], wml[128:])
    p2a, p2b = _prop_edge(src, dst, v2, v2c)
    noise = _pad_rows(jax.random.normal(jax.random.key(42), (N, 64), jnp.float32))
    mean, logvar, z, v3a = _tc_c(p2a, p2b, v2, dinv, noise, bml, W_z2h)
    y3a, y3b = _prop_col(src, dst, v3a, v3b)
    v4, v4c = _tc_d(y3a, y3b, dinv, b_z2h.reshape(1, -1), b_cd.reshape(1, -1),
                    W_out[:128], W_out[128:])
    p4a, p4b = _prop_edge(src, dst, v4, v4c)
    out = _tc_e(p4a, p4b, v4, dinv, b_out.reshape(1, -1))
    return (z[:N], mean[:N], logvar[:N], out[:N])


# speedup vs baseline: 1.1540x; 1.0192x over previous
import jax
import jax.numpy as jnp
from jax import lax
from jax.experimental import pallas as pl
from jax.experimental.pallas import tpu as pltpu
from jax.experimental.pallas import tpu_sc as plsc

N = 10000
NP = 10240
E = 320000
K = 128
ROWSP = 2560
EP = ROWSP * K
NSUB = 16
NW = 32
SB = 40
RPS = ROWSP // NSUB
RPSE = ROWSP // NW
NPS = NP // NSUB
EPD = 327680
ECH = 2560
EPW = EPD // NW

_MESH = plsc.VectorSubcoreMesh(core_axis_name="c", subcore_axis_name="s")


def _deg_body(dstf, degp, hist, dbuf):
    cid = lax.axis_index("c")
    sid = lax.axis_index("s")
    wid = cid * NSUB + sid

    def zero(i, c):
        hist[pl.ds(i * 16, 16)] = jnp.zeros((16,), jnp.float32)
        return c

    lax.fori_loop(0, NP // 16, zero, 0)
    base = wid * EPW
    ones = jnp.ones((16,), jnp.float32)

    def outer(g, carry):
        pltpu.sync_copy(dstf.at[pl.ds(base + g * ECH, ECH)], dbuf)

        def inner(p, c2):
            idx = dbuf[pl.ds(p * 16, 16)]
            plsc.addupdate_scatter(hist, [idx], ones)
            return c2

        return lax.fori_loop(0, ECH // 16, inner, carry)

    lax.fori_loop(0, EPW // ECH, outer, 0)
    pltpu.sync_copy(hist, degp.at[pl.ds(wid * NP, NP)])


_deg_kernel = pl.kernel(
    _deg_body,
    out_type=jax.ShapeDtypeStruct((NW * NP,), jnp.float32),
    mesh=_MESH,
    compiler_params=pltpu.CompilerParams(needs_layout_passes=False),
    scratch_types=[
        pltpu.VMEM((NP,), jnp.float32),
        pltpu.VMEM((ECH,), jnp.int32),
    ],
)


ND = 2


def _edge_sweep(src2, dst2, v, acc, sidx, didx, rows, semg, sems,
                row0, nrefill):
    def outer(g, carry):
        pltpu.sync_copy(src2.at[pl.ds(row0 + g * SB, SB)], sidx)
        pltpu.sync_copy(dst2.at[pl.ds(row0 + g * SB, SB)], didx)
        cp = pltpu.async_copy(v.at[sidx.at[0]], rows[0], semg[0])
        for j in range(1, SB):
            cp_next = pltpu.async_copy(v.at[sidx.at[j]], rows[j % 2],
                                       semg[j % 2])
            cp.wait()
            pltpu.sync_copy(rows[(j - 1) % 2], acc.at[didx.at[j - 1]],
                            add=True)
            cp = cp_next
        cp.wait()
        pltpu.sync_copy(rows[(SB - 1) % 2], acc.at[didx.at[SB - 1]], add=True)
        return carry

    lax.fori_loop(0, nrefill, outer, 0)


_PROP_SCRATCH = (
    [pltpu.VMEM_SHARED((NP, 128), jnp.float32),
     pltpu.VMEM((SB, K), jnp.int32),
     pltpu.VMEM((SB, K), jnp.int32)]
    + [pltpu.VMEM((K, 128), jnp.float32) for _ in range(ND)]
    + [pltpu.SemaphoreType.DMA for _ in range(2 * ND)]
)

_PROP_OUT = [jax.ShapeDtypeStruct((NP, 128), jnp.float32),
             jax.ShapeDtypeStruct((NP, 128), jnp.float32)]


def _prop_col_body(src2, dst2, va, vb, ya, yb, acc, sidx, didx, *bufs):
    rows, semg, sems = bufs[:ND], bufs[ND:2 * ND], bufs[2 * ND:]
    cid = lax.axis_index("c")
    sid = lax.axis_index("s")

    def run(v, y):
        pltpu.sync_copy(v.at[pl.ds(sid * NPS, NPS)],
                        acc.at[pl.ds(sid * NPS, NPS)])
        plsc.subcore_barrier()
        _edge_sweep(src2, dst2, v, acc, sidx, didx, rows,
                    semg, sems, sid * RPS, RPS // SB)
        plsc.subcore_barrier()
        pltpu.sync_copy(acc.at[pl.ds(sid * NPS, NPS)],
                        y.at[pl.ds(sid * NPS, NPS)])

    @pl.when(cid == 0)
    def _():
        run(va, ya)

    @pl.when(cid == 1)
    def _():
        run(vb, yb)


_prop_col = pl.kernel(
    _prop_col_body,
    out_type=_PROP_OUT,
    mesh=_MESH,
    scratch_types=_PROP_SCRATCH,
)


def _prop_edge_body(src2, dst2, va, vb, pa, pb, acc, sidx, didx, *bufs):
    rows, semg, sems = bufs[:ND], bufs[ND:2 * ND], bufs[2 * ND:]
    cid = lax.axis_index("c")
    sid = lax.axis_index("s")

    def run(v, p):
        def zrow(r, c):
            for cc in range(8):
                rows[0][r, pl.ds(cc * 16, 16)] = jnp.zeros((16,), jnp.float32)
            return c

        lax.fori_loop(0, K, zrow, 0)
        for i in range(NPS // K):
            pltpu.sync_copy(rows[0], acc.at[pl.ds(sid * NPS + i * K, K)])
        plsc.subcore_barrier()
        _edge_sweep(src2, dst2, v, acc, sidx, didx, rows,
                    semg, sems, (cid * NSUB + sid) * RPSE, RPSE // SB)
        plsc.subcore_barrier()
        pltpu.sync_copy(acc.at[pl.ds(sid * NPS, NPS)],
                        p.at[pl.ds(sid * NPS, NPS)])

    @pl.when(cid == 0)
    def _():
        run(va, pa)

    @pl.when(cid == 1)
    def _():
        run(vb, pb)


_prop_edge = pl.kernel(
    _prop_edge_body,
    out_type=_PROP_OUT,
    mesh=_MESH,
    scratch_types=_PROP_SCRATCH,
)


_R = 1024
_G = NP // _R


def _row_spec(c):
    return pl.BlockSpec((_R, c), lambda i: (i, 0))


def _full_spec(r, c):
    return pl.BlockSpec((r, c), lambda i: (0, 0))


def _tc_a_body(dgt, f, c, wf, wce, wcd, dinv_o, v1a_o, v1b_o, v3b_o):
    deg = jnp.sum(dgt[...], axis=1, keepdims=True) + 1.0
    dinv = lax.rsqrt(deg)
    dinv_o[...] = dinv
    v1a_o[...] = dinv * jnp.dot(f[...], wf[...], preferred_element_type=jnp.float32)
    cc = c[...]
    v1b_o[...] = dinv * jnp.dot(cc, wce[...], preferred_element_type=jnp.float32)
    v3b_o[...] = dinv * jnp.dot(cc, wcd[...], preferred_element_type=jnp.float32)


_tc_a = pl.pallas_call(
    _tc_a_body,
    grid=(_G,),
    in_specs=[_row_spec(NW), _row_spec(128), _row_spec(16),
              _full_spec(128, 128), _full_spec(16, 128), _full_spec(16, 128)],
    out_specs=[_row_spec(1), _row_spec(128), _row_spec(128), _row_spec(128)],
    out_shape=[jax.ShapeDtypeStruct((NP, 1), jnp.float32),
               jax.ShapeDtypeStruct((NP, 128), jnp.float32),
               jax.ShapeDtypeStruct((NP, 128), jnp.float32),
               jax.ShapeDtypeStruct((NP, 128), jnp.float32)],
)


def _tc_b_body(ya, yb, dv, bf, bc, wt, wb, v2_o, v2c_o):
    dinv = dv[...]
    ha = jnp.tanh(dinv * ya[...] + bf[...])
    hb = jnp.tanh(dinv * yb[...] + bc[...])
    v2 = dinv * (jnp.dot(ha, wt[...], preferred_element_type=jnp.float32)
                 + jnp.dot(hb, wb[...], preferred_element_type=jnp.float32))
    v2_o[...] = v2
    v2c_o[...] = v2


_tc_b = pl.pallas_call(
    _tc_b_body,
    grid=(_G,),
    in_specs=[_row_spec(128), _row_spec(128), _row_spec(1),
              _full_spec(1, 128), _full_spec(1, 128),
              _full_spec(128, 128), _full_spec(128, 128)],
    out_specs=[_row_spec(128), _row_spec(128)],
    out_shape=[jax.ShapeDtypeStruct((NP, 128), jnp.float32),
               jax.ShapeDtypeStruct((NP, 128), jnp.float32)],
)


def _tc_c_body(pa, pb, v2, dv, nz, bml, wz, mean_o, logvar_o, z_o, v3a_o):
    dinv = dv[...]
    y2 = dinv * (pa[...] + pb[...] + v2[...]) + bml[...]
    mean = y2[:, :64]
    logvar = y2[:, 64:]
    z = nz[...] * jnp.exp(0.5 * logvar) + mean
    mean_o[...] = mean
    logvar_o[...] = logvar
    z_o[...] = z
    v3a_o[...] = dinv * jnp.dot(z, wz[...], preferred_element_type=jnp.float32)


_tc_c = pl.pallas_call(
    _tc_c_body,
    grid=(_G,),
    in_specs=[_row_spec(128), _row_spec(128), _row_spec(128), _row_spec(1),
              _row_spec(64), _full_spec(1, 128), _full_spec(64, 128)],
    out_specs=[_row_spec(64), _row_spec(64), _row_spec(64), _row_spec(128)],
    out_shape=[jax.ShapeDtypeStruct((NP, 64), jnp.float32),
               jax.ShapeDtypeStruct((NP, 64), jnp.float32),
               jax.ShapeDtypeStruct((NP, 64), jnp.float32),
               jax.ShapeDtypeStruct((NP, 128), jnp.float32)],
)


def _tc_d_body(ya, yb, dv, bz, bc, wt, wb, v4_o, v4c_o):
    dinv = dv[...]
    ha = jnp.tanh(dinv * ya[...] + bz[...])
    hb = jnp.tanh(dinv * yb[...] + bc[...])
    v4 = dinv * (jnp.dot(ha, wt[...], preferred_element_type=jnp.float32)
                 + jnp.dot(hb, wb[...], preferred_element_type=jnp.float32))
    v4_o[...] = v4
    v4c_o[...] = v4


_tc_d = pl.pallas_call(
    _tc_d_body,
    grid=(_G,),
    in_specs=[_row_spec(128), _row_spec(128), _row_spec(1),
              _full_spec(1, 128), _full_spec(1, 128),
              _full_spec(128, 128), _full_spec(128, 128)],
    out_specs=[_row_spec(128), _row_spec(128)],
    out_shape=[jax.ShapeDtypeStruct((NP, 128), jnp.float32),
               jax.ShapeDtypeStruct((NP, 128), jnp.float32)],
)


def _tc_e_body(pa, pb, v4, dv, bo, out_o):
    out_o[...] = dv[...] * (pa[...] + pb[...] + v4[...]) + bo[...]


_tc_e = pl.pallas_call(
    _tc_e_body,
    grid=(_G,),
    in_specs=[_row_spec(128), _row_spec(128), _row_spec(128), _row_spec(1),
              _full_spec(1, 128)],
    out_specs=_row_spec(128),
    out_shape=jax.ShapeDtypeStruct((NP, 128), jnp.float32),
)


def _pad_rows(x):
    return jnp.pad(x, ((0, NP - N), (0, 0)))


def kernel(feature, condition, edge_index, W_f2h, b_f2h, W_ce, b_ce, W_mean,
           b_mean, W_logvar, b_logvar, W_z2h, b_z2h, W_cd, b_cd, W_out, b_out):
    src0 = edge_index[0].astype(jnp.int32)
    dst0 = edge_index[1].astype(jnp.int32)
    src = jnp.concatenate(
        [src0, jnp.zeros((EP - E,), jnp.int32)]).reshape(ROWSP, K)
    dst = jnp.concatenate(
        [dst0, jnp.full((EP - E,), N, jnp.int32)]).reshape(ROWSP, K)
    dstd = jnp.concatenate([dst0, jnp.full((EPD - E,), N, jnp.int32)])

    degp = _deg_kernel(dstd)
    degt = degp.reshape(NW, NP).T

    wml = jnp.concatenate([W_mean, W_logvar], axis=1)
    bml = jnp.concatenate([b_mean, b_logvar]).reshape(1, -1)

    featp = _pad_rows(feature)
    condp = _pad_rows(condition)

    dinv, v1a, v1b, v3b = _tc_a(degt, featp, condp, W_f2h, W_ce, W_cd)
    y1a, y1b = _prop_col(src, dst, v1a, v1b)
    v2, v2c = _tc_b(y1a, y1b, dinv, b_f2h.reshape(1, -1), b_ce.reshape(1, -1),
                    wml[:128], wml[128:])
    p2a, p2b = _prop_edge(src, dst, v2, v2c)
    noise = _pad_rows(jax.random.normal(jax.random.key(42), (N, 64), jnp.float32))
    mean, logvar, z, v3a = _tc_c(p2a, p2b, v2, dinv, noise, bml, W_z2h)
    y3a, y3b = _prop_col(src, dst, v3a, v3b)
    v4, v4c = _tc_d(y3a, y3b, dinv, b_z2h.reshape(1, -1), b_cd.reshape(1, -1),
                    W_out[:128], W_out[128:])
    p4a, p4b = _prop_edge(src, dst, v4, v4c)
    out = _tc_e(p4a, p4b, v4, dinv, b_out.reshape(1, -1))
    return (z[:N], mean[:N], logvar[:N], out[:N])


# speedup vs baseline: 1.1548x; 1.0007x over previous
import jax
import jax.numpy as jnp
from jax import lax
from jax.experimental import pallas as pl
from jax.experimental.pallas import tpu as pltpu
from jax.experimental.pallas import tpu_sc as plsc

N = 10000
NP = 10240
E = 320000
K = 128
ROWSP = 2560
EP = ROWSP * K
NSUB = 16
NW = 32
SB = 40
RPS = ROWSP // NSUB
RPSE = ROWSP // NW
NPS = NP // NSUB
EPD = 327680
ECH = 2560
EPW = EPD // NW

_MESH = plsc.VectorSubcoreMesh(core_axis_name="c", subcore_axis_name="s")


def _deg_body(dstf, degp, hist, dbuf):
    cid = lax.axis_index("c")
    sid = lax.axis_index("s")
    wid = cid * NSUB + sid

    def zero(i, c):
        hist[pl.ds(i * 16, 16)] = jnp.zeros((16,), jnp.float32)
        return c

    lax.fori_loop(0, NP // 16, zero, 0)
    base = wid * EPW
    ones = jnp.ones((16,), jnp.float32)

    def outer(g, carry):
        pltpu.sync_copy(dstf.at[pl.ds(base + g * ECH, ECH)], dbuf)

        def inner(p, c2):
            idx = dbuf[pl.ds(p * 16, 16)]
            plsc.addupdate_scatter(hist, [idx], ones)
            return c2

        return lax.fori_loop(0, ECH // 16, inner, carry)

    lax.fori_loop(0, EPW // ECH, outer, 0)
    pltpu.sync_copy(hist, degp.at[pl.ds(wid * NP, NP)])


_deg_kernel = pl.kernel(
    _deg_body,
    out_type=jax.ShapeDtypeStruct((NW * NP,), jnp.float32),
    mesh=_MESH,
    compiler_params=pltpu.CompilerParams(needs_layout_passes=False),
    scratch_types=[
        pltpu.VMEM((NP,), jnp.float32),
        pltpu.VMEM((ECH,), jnp.int32),
    ],
)


ND = 2


def _edge_sweep(src2, dst2, v, acc, sidx, didx, rows, semg, sems,
                row0, nrefill):
    def outer(g, carry):
        pltpu.sync_copy(src2.at[pl.ds(row0 + g * SB, SB)], sidx)
        pltpu.sync_copy(dst2.at[pl.ds(row0 + g * SB, SB)], didx)
        cp = pltpu.async_copy(v.at[sidx.at[0]], rows[0], semg[0])
        for j in range(1, SB):
            cp_next = pltpu.async_copy(v.at[sidx.at[j]], rows[j % 2],
                                       semg[j % 2])
            cp.wait()
            pltpu.sync_copy(rows[(j - 1) % 2], acc.at[didx.at[j - 1]],
                            add=True)
            cp = cp_next
        cp.wait()
        pltpu.sync_copy(rows[(SB - 1) % 2], acc.at[didx.at[SB - 1]], add=True)
        return carry

    lax.fori_loop(0, nrefill, outer, 0)


_PROP_SCRATCH = (
    [pltpu.VMEM_SHARED((NP, 128), jnp.float32),
     pltpu.VMEM((SB, K), jnp.int32),
     pltpu.VMEM((SB, K), jnp.int32)]
    + [pltpu.VMEM((K, 128), jnp.float32) for _ in range(ND)]
    + [pltpu.SemaphoreType.DMA for _ in range(2 * ND)]
)

_PROP_OUT = [jax.ShapeDtypeStruct((NP, 128), jnp.float32),
             jax.ShapeDtypeStruct((NP, 128), jnp.float32)]


def _prop_col_body(src2, dst2, va, vb, ya, yb, acc, sidx, didx, *bufs):
    rows, semg, sems = bufs[:ND], bufs[ND:2 * ND], bufs[2 * ND:]
    cid = lax.axis_index("c")
    sid = lax.axis_index("s")

    def run(v, y):
        pltpu.sync_copy(v.at[pl.ds(sid * NPS, NPS)],
                        acc.at[pl.ds(sid * NPS, NPS)])
        plsc.subcore_barrier()
        _edge_sweep(src2, dst2, v, acc, sidx, didx, rows,
                    semg, sems, sid * RPS, RPS // SB)
        plsc.subcore_barrier()
        pltpu.sync_copy(acc.at[pl.ds(sid * NPS, NPS)],
                        y.at[pl.ds(sid * NPS, NPS)])

    @pl.when(cid == 0)
    def _():
        run(va, ya)

    @pl.when(cid == 1)
    def _():
        run(vb, yb)


_prop_col = pl.kernel(
    _prop_col_body,
    out_type=_PROP_OUT,
    mesh=_MESH,
    scratch_types=_PROP_SCRATCH,
)


def _prop_edge_body(src2, dst2, va, vb, pa, pb, acc, sidx, didx, *bufs):
    rows, semg, sems = bufs[:ND], bufs[ND:2 * ND], bufs[2 * ND:]
    cid = lax.axis_index("c")
    sid = lax.axis_index("s")

    def run(v, p):
        def zrow(r, c):
            for cc in range(8):
                rows[0][r, pl.ds(cc * 16, 16)] = jnp.zeros((16,), jnp.float32)
            return c

        lax.fori_loop(0, K, zrow, 0)
        for i in range(NPS // K):
            pltpu.sync_copy(rows[0], acc.at[pl.ds(sid * NPS + i * K, K)])
        plsc.subcore_barrier()
        _edge_sweep(src2, dst2, v, acc, sidx, didx, rows,
                    semg, sems, (sid * 2 + cid) * RPSE, RPSE // SB)
        plsc.subcore_barrier()
        pltpu.sync_copy(acc.at[pl.ds(sid * NPS, NPS)],
                        p.at[pl.ds(sid * NPS, NPS)])

    @pl.when(cid == 0)
    def _():
        run(va, pa)

    @pl.when(cid == 1)
    def _():
        run(vb, pb)


_prop_edge = pl.kernel(
    _prop_edge_body,
    out_type=_PROP_OUT,
    mesh=_MESH,
    scratch_types=_PROP_SCRATCH,
)


_R = 1024
_G = NP // _R


def _row_spec(c):
    return pl.BlockSpec((_R, c), lambda i: (i, 0))


def _full_spec(r, c):
    return pl.BlockSpec((r, c), lambda i: (0, 0))


def _tc_a_body(dgt, f, c, wf, wce, wcd, dinv_o, v1a_o, v1b_o, v3b_o):
    deg = jnp.sum(dgt[...], axis=1, keepdims=True) + 1.0
    dinv = lax.rsqrt(deg)
    dinv_o[...] = dinv
    v1a_o[...] = dinv * jnp.dot(f[...], wf[...], preferred_element_type=jnp.float32)
    cc = c[...]
    v1b_o[...] = dinv * jnp.dot(cc, wce[...], preferred_element_type=jnp.float32)
    v3b_o[...] = dinv * jnp.dot(cc, wcd[...], preferred_element_type=jnp.float32)


_tc_a = pl.pallas_call(
    _tc_a_body,
    grid=(_G,),
    in_specs=[_row_spec(NW), _row_spec(128), _row_spec(16),
              _full_spec(128, 128), _full_spec(16, 128), _full_spec(16, 128)],
    out_specs=[_row_spec(1), _row_spec(128), _row_spec(128), _row_spec(128)],
    out_shape=[jax.ShapeDtypeStruct((NP, 1), jnp.float32),
               jax.ShapeDtypeStruct((NP, 128), jnp.float32),
               jax.ShapeDtypeStruct((NP, 128), jnp.float32),
               jax.ShapeDtypeStruct((NP, 128), jnp.float32)],
)


def _tc_b_body(ya, yb, dv, bf, bc, wt, wb, v2_o, v2c_o):
    dinv = dv[...]
    ha = jnp.tanh(dinv * ya[...] + bf[...])
    hb = jnp.tanh(dinv * yb[...] + bc[...])
    v2 = dinv * (jnp.dot(ha, wt[...], preferred_element_type=jnp.float32)
                 + jnp.dot(hb, wb[...], preferred_element_type=jnp.float32))
    v2_o[...] = v2
    v2c_o[...] = v2


_tc_b = pl.pallas_call(
    _tc_b_body,
    grid=(_G,),
    in_specs=[_row_spec(128), _row_spec(128), _row_spec(1),
              _full_spec(1, 128), _full_spec(1, 128),
              _full_spec(128, 128), _full_spec(128, 128)],
    out_specs=[_row_spec(128), _row_spec(128)],
    out_shape=[jax.ShapeDtypeStruct((NP, 128), jnp.float32),
               jax.ShapeDtypeStruct((NP, 128), jnp.float32)],
)


def _tc_c_body(pa, pb, v2, dv, nz, bml, wz, mean_o, logvar_o, z_o, v3a_o):
    dinv = dv[...]
    y2 = dinv * (pa[...] + pb[...] + v2[...]) + bml[...]
    mean = y2[:, :64]
    logvar = y2[:, 64:]
    z = nz[...] * jnp.exp(0.5 * logvar) + mean
    mean_o[...] = mean
    logvar_o[...] = logvar
    z_o[...] = z
    v3a_o[...] = dinv * jnp.dot(z, wz[...], preferred_element_type=jnp.float32)


_tc_c = pl.pallas_call(
    _tc_c_body,
    grid=(_G,),
    in_specs=[_row_spec(128), _row_spec(128), _row_spec(128), _row_spec(1),
              _row_spec(64), _full_spec(1, 128), _full_spec(64, 128)],
    out_specs=[_row_spec(64), _row_spec(64), _row_spec(64), _row_spec(128)],
    out_shape=[jax.ShapeDtypeStruct((NP, 64), jnp.float32),
               jax.ShapeDtypeStruct((NP, 64), jnp.float32),
               jax.ShapeDtypeStruct((NP, 64), jnp.float32),
               jax.ShapeDtypeStruct((NP, 128), jnp.float32)],
)


def _tc_d_body(ya, yb, dv, bz, bc, wt, wb, v4_o, v4c_o):
    dinv = dv[...]
    ha = jnp.tanh(dinv * ya[...] + bz[...])
    hb = jnp.tanh(dinv * yb[...] + bc[...])
    v4 = dinv * (jnp.dot(ha, wt[...], preferred_element_type=jnp.float32)
                 + jnp.dot(hb, wb[...], preferred_element_type=jnp.float32))
    v4_o[...] = v4
    v4c_o[...] = v4


_tc_d = pl.pallas_call(
    _tc_d_body,
    grid=(_G,),
    in_specs=[_row_spec(128), _row_spec(128), _row_spec(1),
              _full_spec(1, 128), _full_spec(1, 128),
              _full_spec(128, 128), _full_spec(128, 128)],
    out_specs=[_row_spec(128), _row_spec(128)],
    out_shape=[jax.ShapeDtypeStruct((NP, 128), jnp.float32),
               jax.ShapeDtypeStruct((NP, 128), jnp.float32)],
)


def _tc_e_body(pa, pb, v4, dv, bo, out_o):
    out_o[...] = dv[...] * (pa[...] + pb[...] + v4[...]) + bo[...]


_tc_e = pl.pallas_call(
    _tc_e_body,
    grid=(_G,),
    in_specs=[_row_spec(128), _row_spec(128), _row_spec(128), _row_spec(1),
              _full_spec(1, 128)],
    out_specs=_row_spec(128),
    out_shape=jax.ShapeDtypeStruct((NP, 128), jnp.float32),
)


def _pad_rows(x):
    return jnp.pad(x, ((0, NP - N), (0, 0)))


def kernel(feature, condition, edge_index, W_f2h, b_f2h, W_ce, b_ce, W_mean,
           b_mean, W_logvar, b_logvar, W_z2h, b_z2h, W_cd, b_cd, W_out, b_out):
    src0 = edge_index[0].astype(jnp.int32)
    dst0 = edge_index[1].astype(jnp.int32)
    src = jnp.concatenate(
        [src0, jnp.zeros((EP - E,), jnp.int32)]).reshape(ROWSP, K)
    dst = jnp.concatenate(
        [dst0, jnp.full((EP - E,), N, jnp.int32)]).reshape(ROWSP, K)
    dstd = jnp.concatenate([dst0, jnp.full((EPD - E,), N, jnp.int32)])

    degp = _deg_kernel(dstd)
    degt = degp.reshape(NW, NP).T

    wml = jnp.concatenate([W_mean, W_logvar], axis=1)
    bml = jnp.concatenate([b_mean, b_logvar]).reshape(1, -1)

    featp = _pad_rows(feature)
    condp = _pad_rows(condition)

    dinv, v1a, v1b, v3b = _tc_a(degt, featp, condp, W_f2h, W_ce, W_cd)
    y1a, y1b = _prop_col(src, dst, v1a, v1b)
    v2, v2c = _tc_b(y1a, y1b, dinv, b_f2h.reshape(1, -1), b_ce.reshape(1, -1),
                    wml[:128], wml[128:])
    p2a, p2b = _prop_edge(src, dst, v2, v2c)
    noise = _pad_rows(jax.random.normal(jax.random.key(42), (N, 64), jnp.float32))
    mean, logvar, z, v3a = _tc_c(p2a, p2b, v2, dinv, noise, bml, W_z2h)
    y3a, y3b = _prop_col(src, dst, v3a, v3b)
    v4, v4c = _tc_d(y3a, y3b, dinv, b_z2h.reshape(1, -1), b_cd.reshape(1, -1),
                    W_out[:128], W_out[128:])
    p4a, p4b = _prop_edge(src, dst, v4, v4c)
    out = _tc_e(p4a, p4b, v4, dinv, b_out.reshape(1, -1))
    return (z[:N], mean[:N], logvar[:N], out[:N])
